# baseline (speedup 1.0000x reference)
import functools

import jax
import jax.numpy as jnp
from jax.experimental import pallas as pl
from jax.experimental.pallas import tpu as pltpu

B = 16
S = 2048
D_IN = 512
D_OUT = 512


G = 1


def _body(x_ref, wsT_ref, wp_ref, bp_ref, o_ref, s_scr):
    xb = x_ref[...]
    x3 = xb.reshape(G * S // 128, 128, D_IN)
    s_scr[...] = jnp.sum(x3 * wsT_ref[...][None], axis=2)
    s = s_scr[...].reshape(G, S // 128, 128)
    mean = jnp.sum(s, axis=(1, 2), keepdims=True) / S
    sumsq = jnp.sum(s * s, axis=(1, 2), keepdims=True)
    var = (sumsq - S * mean * mean) / (S - 1)
    a = 1.0 / (jnp.sqrt(var) + 1e-7)
    e = jnp.exp(s * a - mean * a)
    w = e.reshape(1, G * S)
    col_g = jax.lax.broadcasted_iota(jnp.int32, (G, G * S), 1) // S
    row_g = jax.lax.broadcasted_iota(jnp.int32, (G, G * S), 0)
    w_bd = jnp.where(col_g == row_g, w, 0.0)
    denom = jnp.sum(e, axis=(1, 2)).reshape(G, 1)
    pooled = jnp.dot(w_bd, xb, preferred_element_type=jnp.float32) / denom
    o_ref[...] = (
        jnp.dot(pooled, wp_ref[...], preferred_element_type=jnp.float32)
        + bp_ref[...]
    )[None]


@jax.jit
def _run(x, wsT, W_proj, bp):
    return pl.pallas_call(
        _body,
        grid=(B // G,),
        in_specs=[
            pl.BlockSpec((G * S, D_IN), lambda b: (b, 0)),
            pl.BlockSpec((1, D_IN), lambda b: (0, 0)),
            pl.BlockSpec((D_IN, D_OUT), lambda b: (0, 0)),
            pl.BlockSpec((1, D_OUT), lambda b: (0, 0)),
        ],
        out_specs=pl.BlockSpec((1, G, D_OUT), lambda b: (b, 0, 0)),
        out_shape=jax.ShapeDtypeStruct((B // G, G, D_OUT), jnp.float32),
        scratch_shapes=[pltpu.VMEM((G * S // 128, 128), jnp.float32)],
        compiler_params=pltpu.CompilerParams(
            dimension_semantics=("arbitrary",),
        ),
    )(x, wsT, W_proj, bp)


def kernel(x, W_proj, b_proj, W_score, b_score, graph_size_list, edge_list):
    wsT = W_score.reshape(1, D_IN)
    bp = b_proj.reshape(1, D_OUT)
    return _run(x, wsT, W_proj, bp).reshape(B, D_OUT)


# speedup vs baseline: 1.1465x; 1.1465x over previous
import functools

import jax
import jax.numpy as jnp
from jax.experimental import pallas as pl
from jax.experimental.pallas import tpu as pltpu

B = 16
S = 2048
D_IN = 512
D_OUT = 512


G = 4


def _body(x_ref, wsT_ref, wp_ref, bp_ref, o_ref, s_scr):
    xb = x_ref[...]
    x3 = xb.reshape(G * S // 128, 128, D_IN)
    s_scr[...] = jnp.sum(x3 * wsT_ref[...][None], axis=2)
    s = s_scr[...].reshape(G, S // 128, 128)
    mean = jnp.sum(s, axis=(1, 2), keepdims=True) / S
    sumsq = jnp.sum(s * s, axis=(1, 2), keepdims=True)
    var = (sumsq - S * mean * mean) / (S - 1)
    a = 1.0 / (jnp.sqrt(var) + 1e-7)
    e = jnp.exp(s * a - mean * a)
    w = e.reshape(1, G * S)
    col_g = jax.lax.broadcasted_iota(jnp.int32, (G, G * S), 1) // S
    row_g = jax.lax.broadcasted_iota(jnp.int32, (G, G * S), 0)
    w_bd = jnp.where(col_g == row_g, w, 0.0)
    denom = jnp.sum(e, axis=(1, 2)).reshape(G, 1)
    pooled = jnp.dot(w_bd, xb, preferred_element_type=jnp.float32) / denom
    o_ref[...] = (
        jnp.dot(pooled, wp_ref[...], preferred_element_type=jnp.float32)
        + bp_ref[...]
    )[None]


@jax.jit
def _run(x, wsT, W_proj, bp):
    return pl.pallas_call(
        _body,
        grid=(B // G,),
        in_specs=[
            pl.BlockSpec((G * S, D_IN), lambda b: (b, 0)),
            pl.BlockSpec((1, D_IN), lambda b: (0, 0)),
            pl.BlockSpec((D_IN, D_OUT), lambda b: (0, 0)),
            pl.BlockSpec((1, D_OUT), lambda b: (0, 0)),
        ],
        out_specs=pl.BlockSpec((1, G, D_OUT), lambda b: (b, 0, 0)),
        out_shape=jax.ShapeDtypeStruct((B // G, G, D_OUT), jnp.float32),
        scratch_shapes=[pltpu.VMEM((G * S // 128, 128), jnp.float32)],
        compiler_params=pltpu.CompilerParams(
            dimension_semantics=("arbitrary",),
        ),
    )(x, wsT, W_proj, bp)


def kernel(x, W_proj, b_proj, W_score, b_score, graph_size_list, edge_list):
    wsT = W_score.reshape(1, D_IN)
    bp = b_proj.reshape(1, D_OUT)
    return _run(x, wsT, W_proj, bp).reshape(B, D_OUT)


# speedup vs baseline: 1.2067x; 1.0524x over previous
import functools

import jax
import jax.numpy as jnp
from jax.experimental import pallas as pl
from jax.experimental.pallas import tpu as pltpu

B = 16
S = 2048
D_IN = 512
D_OUT = 512


G = 2


def _body(x_ref, wsT_ref, wp_ref, bp_ref, o_ref, s_scr):
    xb = x_ref[...]
    x3 = xb.reshape(G * S // 128, 128, D_IN)
    s_scr[...] = jnp.sum(x3 * wsT_ref[...][None], axis=2)
    s = s_scr[...].reshape(G, S // 128, 128)
    mean = jnp.sum(s, axis=(1, 2), keepdims=True) / S
    sumsq = jnp.sum(s * s, axis=(1, 2), keepdims=True)
    var = (sumsq - S * mean * mean) / (S - 1)
    a = 1.0 / (jnp.sqrt(var) + 1e-7)
    e = jnp.exp(s * a - mean * a)
    w = e.reshape(1, G * S)
    col_g = jax.lax.broadcasted_iota(jnp.int32, (G, G * S), 1) // S
    row_g = jax.lax.broadcasted_iota(jnp.int32, (G, G * S), 0)
    w_bd = jnp.where(col_g == row_g, w, 0.0)
    denom = jnp.sum(e, axis=(1, 2)).reshape(G, 1)
    pooled = jnp.dot(w_bd, xb, preferred_element_type=jnp.float32) / denom
    o_ref[...] = (
        jnp.dot(pooled, wp_ref[...], preferred_element_type=jnp.float32)
        + bp_ref[...]
    )[None]


@jax.jit
def _run(x, wsT, W_proj, bp):
    return pl.pallas_call(
        _body,
        grid=(B // G,),
        in_specs=[
            pl.BlockSpec((G * S, D_IN), lambda b: (b, 0)),
            pl.BlockSpec((1, D_IN), lambda b: (0, 0)),
            pl.BlockSpec((D_IN, D_OUT), lambda b: (0, 0)),
            pl.BlockSpec((1, D_OUT), lambda b: (0, 0)),
        ],
        out_specs=pl.BlockSpec((1, G, D_OUT), lambda b: (b, 0, 0)),
        out_shape=jax.ShapeDtypeStruct((B // G, G, D_OUT), jnp.float32),
        scratch_shapes=[pltpu.VMEM((G * S // 128, 128), jnp.float32)],
        compiler_params=pltpu.CompilerParams(
            dimension_semantics=("arbitrary",),
        ),
    )(x, wsT, W_proj, bp)


def kernel(x, W_proj, b_proj, W_score, b_score, graph_size_list, edge_list):
    wsT = W_score.reshape(1, D_IN)
    bp = b_proj.reshape(1, D_OUT)
    return _run(x, wsT, W_proj, bp).reshape(B, D_OUT)
